# Initial kernel scaffold; baseline (speedup 1.0000x reference)
import jax
import jax.numpy as jnp
from jax.experimental import pallas as pl


def kernel(triplets, ent_embed, rel_embed, adj, a_w, a_b, a2_w, a2_b):
    raise NotImplementedError("write your pallas kernel here")


# speedup vs baseline: 3.5976x; 3.5976x over previous
import functools

import jax
import jax.numpy as jnp
from jax import lax
from jax.experimental import pallas as pl
from jax.experimental.pallas import tpu as pltpu
from jax.experimental.pallas import tpu_sc as plsc

HALF = 128


def _a_body(ent_ref, rel_ref, aw_ref, ab_ref, a2_ref, p1_ref, t2_ref, t3_ref,
            q_ref):
    e = ent_ref[...]
    r = rel_ref[...]
    aw = aw_ref[...]
    hi = lax.Precision.HIGHEST
    dn = (((1,), (1,)), ((), ()))
    c1 = lax.dot_general(e, aw[:, 0:256], dn, precision=hi,
                         preferred_element_type=jnp.float32) + ab_ref[...]
    c2 = lax.dot_general(r, aw[:, 256:512], dn, precision=hi,
                         preferred_element_type=jnp.float32)
    c3 = lax.dot_general(e, aw[:, 512:768], dn, precision=hi,
                         preferred_element_type=jnp.float32)
    a2 = a2_ref[...]
    q1 = lax.dot_general(c1, a2, dn, precision=hi,
                         preferred_element_type=jnp.float32)
    q2 = lax.dot_general(c2, a2, dn, precision=hi,
                         preferred_element_type=jnp.float32)
    q3 = lax.dot_general(c3, a2, dn, precision=hi,
                         preferred_element_type=jnp.float32)
    q_ref[...] = jnp.concatenate([q1, q2, q3, jnp.zeros_like(q1)], axis=1)
    p1_ref[...] = c1
    t2_ref[0] = c2[:, :HALF]
    t2_ref[1] = c2[:, HALF:]
    t3_ref[0] = c3[:, :HALF]
    t3_ref[1] = c3[:, HALF:]


def _stage_a(ent, rel, a_w, a_b2d, a2_w):
    n, d = ent.shape
    bn = 2000
    grid = (n // bn,)
    return pl.pallas_call(
        _a_body,
        grid=grid,
        in_specs=[
            pl.BlockSpec((bn, d), lambda i: (i, 0)),
            pl.BlockSpec((bn, d), lambda i: (i, 0)),
            pl.BlockSpec((d, 3 * d), lambda i: (0, 0)),
            pl.BlockSpec((1, d), lambda i: (0, 0)),
            pl.BlockSpec((1, d), lambda i: (0, 0)),
        ],
        out_specs=[
            pl.BlockSpec((bn, d), lambda i: (i, 0)),
            pl.BlockSpec((2, bn, HALF), lambda i: (0, i, 0)),
            pl.BlockSpec((2, bn, HALF), lambda i: (0, i, 0)),
            pl.BlockSpec((bn, 4), lambda i: (i, 0)),
        ],
        out_shape=[
            jax.ShapeDtypeStruct((n, d), jnp.float32),
            jax.ShapeDtypeStruct((2, n, HALF), jnp.float32),
            jax.ShapeDtypeStruct((2, n, HALF), jnp.float32),
            jax.ShapeDtypeStruct((n, 4), jnp.float32),
        ],
    )(ent, rel, a_w, a_b2d, a2_w)


def _make_stage_b(n_ent, n_edges):
    NC, NS, L = 2, 16, 16
    C = 80
    epw = n_edges // NS
    nch = epw // C
    RB = 200
    nblk = n_ent // RB
    nrb = -(-nblk // NS)
    mesh = plsc.VectorSubcoreMesh(core_axis_name="c", subcore_axis_name="s")

    @functools.partial(
        pl.kernel,
        mesh=mesh,
        out_type=[jax.ShapeDtypeStruct((2, n_ent, HALF), jnp.float32),
                  jax.ShapeDtypeStruct((n_ent,), jnp.float32)],
        scratch_types=[
            pltpu.VMEM((C,), jnp.float32),
            pltpu.VMEM((C,), jnp.float32),
            pltpu.VMEM((C,), jnp.float32),
            pltpu.VMEM((C,), jnp.int32),
            pltpu.VMEM((C,), jnp.int32),
            pltpu.VMEM((C,), jnp.int32),
            pltpu.VMEM((C,), jnp.int32),
            pltpu.VMEM((C,), jnp.int32),
            pltpu.VMEM((C,), jnp.float32),
            pltpu.VMEM((C, HALF), jnp.float32),
            pltpu.VMEM((C, HALF), jnp.float32),
            pltpu.VMEM((RB, HALF), jnp.float32),
            pltpu.VMEM((RB,), jnp.float32),
            pltpu.VMEM_SHARED((n_ent, HALF), jnp.float32),
            pltpu.VMEM_SHARED((n_ent,), jnp.float32),
            pltpu.SemaphoreType.DMA,
            pltpu.SemaphoreType.DMA,
            pltpu.SemaphoreType.DMA,
        ],
    )
    def stage_b(h_hbm, r_hbm, t_hbm, q1_hbm, q2_hbm, q3_hbm,
                t2_hbm, t3_hbm, s_hbm, d_hbm,
                q1g, q2g, q3g, hv, rv, tv, rfv, tfv, wv, r2v, r3v,
                zv, zdv, acc, accd, semq, sem2, sem3):
        cid = lax.axis_index("c")
        sid = lax.axis_index("s")

        zvec = jnp.zeros((L,), jnp.float32)

        def _zrow(i, _):
            for j in range(HALF // L):
                zv[i, pl.ds(j * L, L)] = zvec
            return 0

        lax.fori_loop(0, RB, _zrow, 0)
        for j in range(RB // L):
            zdv[pl.ds(j * L, L)] = zvec
        zdv[pl.ds(RB - L, L)] = zvec
        for b in range(nrb):
            m = sid + NS * b

            @pl.when(m < nblk)
            def _():
                pltpu.sync_copy(zv, acc.at[pl.ds(m * RB, RB)])
                pltpu.sync_copy(zdv, accd.at[pl.ds(m * RB, RB)])
        plsc.subcore_barrier()

        base0 = sid * epw
        coff = cid * n_ent

        def _chunk(k, _):
            b = base0 + k * C
            pltpu.sync_copy(h_hbm.at[pl.ds(b, C)], hv)
            pltpu.sync_copy(r_hbm.at[pl.ds(b, C)], rv)
            pltpu.sync_copy(t_hbm.at[pl.ds(b, C)], tv)
            for j in range(C // L):
                sl = pl.ds(j * L, L)
                rfv[sl] = rv[sl] + coff
                tfv[sl] = tv[sl] + coff
            cp2 = pltpu.async_copy(t2_hbm.at[rfv], r2v, sem2)
            cp3 = pltpu.async_copy(t3_hbm.at[tfv], r3v, sem3)
            cq1 = pltpu.async_copy(q1_hbm.at[hv], q1g, semq)
            cq2 = pltpu.async_copy(q2_hbm.at[rv], q2g, semq)
            cq3 = pltpu.async_copy(q3_hbm.at[tv], q3g, semq)
            cq1.wait()
            cq2.wait()
            cq3.wait()
            for j in range(C // L):
                sl = pl.ds(j * L, L)
                s = q1g[sl] + q2g[sl] + q3g[sl]
                wv[sl] = jnp.exp(jnp.maximum(s, 0.01 * s))
            cp2.wait()
            cp3.wait()

            def _grp(g, _):
                w16 = wv[pl.ds(g * L, L)]
                for lane in range(L):
                    e = g * L + lane
                    ws = w16[lane]
                    for j in range(HALF // L):
                        sl = pl.ds(j * L, L)
                        r2v[e, sl] = (r2v[e, sl] + r3v[e, sl]) * ws
                return 0

            lax.fori_loop(0, C // L, _grp, 0)
            pltpu.sync_copy(r2v, acc.at[hv], add=True)

            @pl.when(cid == 0)
            def _():
                pltpu.sync_copy(wv, accd.at[hv], add=True)
            return 0

        lax.fori_loop(0, nch, _chunk, 0)
        plsc.subcore_barrier()
        for b in range(nrb):
            m = sid + NS * b

            @pl.when(m < nblk)
            def _():
                s0 = m * RB
                pltpu.sync_copy(acc.at[pl.ds(s0, RB)],
                                s_hbm.at[cid, pl.ds(s0, RB)])

                @pl.when(cid == 0)
                def _():
                    pltpu.sync_copy(accd.at[pl.ds(s0, RB)], zdv)
                    pltpu.sync_copy(zdv, d_hbm.at[pl.ds(s0, RB)])

    return stage_b


def _c_body(s_ref, d_ref, p1_ref, o_ref):
    d = d_ref[...]
    s = jnp.concatenate([s_ref[0], s_ref[1]], axis=1)
    numer = d * p1_ref[...] + s
    x = numer / jnp.maximum(d, 1e-12)
    o_ref[...] = jnp.where(x > 0, x, jnp.exp(jnp.minimum(x, 0.0)) - 1.0)


def _stage_c(sacc, denom2d, p1):
    n, d = p1.shape
    bn = 2000
    return pl.pallas_call(
        _c_body,
        grid=(n // bn,),
        in_specs=[
            pl.BlockSpec((2, bn, HALF), lambda i: (0, i, 0)),
            pl.BlockSpec((bn, 1), lambda i: (i, 0)),
            pl.BlockSpec((bn, d), lambda i: (i, 0)),
        ],
        out_specs=pl.BlockSpec((bn, d), lambda i: (i, 0)),
        out_shape=jax.ShapeDtypeStruct((n, d), jnp.float32),
    )(sacc, denom2d, p1)


def kernel(triplets, ent_embed, rel_embed, adj, a_w, a_b, a2_w, a2_b):
    del adj
    n, d = ent_embed.shape
    e = triplets.shape[1]
    a_b2d = a_b.reshape(1, d)
    p1, t2, t3, q = _stage_a(ent_embed, rel_embed, a_w, a_b2d, a2_w)
    q1 = q[:, 0] + a2_b[0]
    t2f = t2.reshape(2 * n, HALF)
    t3f = t3.reshape(2 * n, HALF)
    sacc, denom = _make_stage_b(n, e)(triplets[0], triplets[1], triplets[2],
                                      q1, q[:, 1], q[:, 2], t2f, t3f)
    return _stage_c(sacc, denom.reshape(n, 1), p1)


# speedup vs baseline: 4.3977x; 1.2224x over previous
import functools

import jax
import jax.numpy as jnp
from jax import lax
from jax.experimental import pallas as pl
from jax.experimental.pallas import tpu as pltpu
from jax.experimental.pallas import tpu_sc as plsc

HALF = 128


def _a_body(ent_ref, rel_ref, aw_ref, ab_ref, a2_ref, p1_ref, t2_ref, t3_ref,
            q_ref):
    e = ent_ref[...]
    r = rel_ref[...]
    aw = aw_ref[...]
    hi = lax.Precision.HIGHEST
    dn = (((1,), (1,)), ((), ()))
    c1 = lax.dot_general(e, aw[:, 0:256], dn, precision=hi,
                         preferred_element_type=jnp.float32) + ab_ref[...]
    c2 = lax.dot_general(r, aw[:, 256:512], dn, precision=hi,
                         preferred_element_type=jnp.float32)
    c3 = lax.dot_general(e, aw[:, 512:768], dn, precision=hi,
                         preferred_element_type=jnp.float32)
    a2 = a2_ref[...]
    q1 = lax.dot_general(c1, a2, dn, precision=hi,
                         preferred_element_type=jnp.float32)
    q2 = lax.dot_general(c2, a2, dn, precision=hi,
                         preferred_element_type=jnp.float32)
    q3 = lax.dot_general(c3, a2, dn, precision=hi,
                         preferred_element_type=jnp.float32)
    q_ref[...] = jnp.concatenate([q1, q2, q3, jnp.zeros_like(q1)], axis=1)
    p1_ref[...] = c1
    t2_ref[0] = c2[:, :HALF]
    t2_ref[1] = c2[:, HALF:]
    t3_ref[0] = c3[:, :HALF]
    t3_ref[1] = c3[:, HALF:]


def _stage_a(ent, rel, a_w, a_b2d, a2_w):
    n, d = ent.shape
    bn = 2000
    grid = (n // bn,)
    return pl.pallas_call(
        _a_body,
        grid=grid,
        in_specs=[
            pl.BlockSpec((bn, d), lambda i: (i, 0)),
            pl.BlockSpec((bn, d), lambda i: (i, 0)),
            pl.BlockSpec((d, 3 * d), lambda i: (0, 0)),
            pl.BlockSpec((1, d), lambda i: (0, 0)),
            pl.BlockSpec((1, d), lambda i: (0, 0)),
        ],
        out_specs=[
            pl.BlockSpec((bn, d), lambda i: (i, 0)),
            pl.BlockSpec((2, bn, HALF), lambda i: (0, i, 0)),
            pl.BlockSpec((2, bn, HALF), lambda i: (0, i, 0)),
            pl.BlockSpec((bn, 4), lambda i: (i, 0)),
        ],
        out_shape=[
            jax.ShapeDtypeStruct((n, d), jnp.float32),
            jax.ShapeDtypeStruct((2, n, HALF), jnp.float32),
            jax.ShapeDtypeStruct((2, n, HALF), jnp.float32),
            jax.ShapeDtypeStruct((n, 4), jnp.float32),
        ],
    )(ent, rel, a_w, a_b2d, a2_w)


def _make_stage_b(n_ent, n_edges):
    NC, NS, L = 2, 16, 16
    C = 80
    epw = n_edges // NS
    nch = epw // C
    assert nch % 2 == 1
    RB = 80
    nblk = n_ent // RB
    nrb = -(-nblk // NS)
    mesh = plsc.VectorSubcoreMesh(core_axis_name="c", subcore_axis_name="s")

    _buf_types = [
        pltpu.VMEM((C,), jnp.int32),
        pltpu.VMEM((C,), jnp.int32),
        pltpu.VMEM((C,), jnp.int32),
        pltpu.VMEM((C,), jnp.int32),
        pltpu.VMEM((C,), jnp.int32),
        pltpu.VMEM((C,), jnp.float32),
        pltpu.VMEM((C,), jnp.float32),
        pltpu.VMEM((C,), jnp.float32),
        pltpu.VMEM((C,), jnp.float32),
        pltpu.VMEM((C, HALF), jnp.float32),
        pltpu.VMEM((C, HALF), jnp.float32),
        pltpu.SemaphoreType.DMA,
        pltpu.SemaphoreType.DMA,
        pltpu.SemaphoreType.DMA,
    ]
    NB = len(_buf_types)

    @functools.partial(
        pl.kernel,
        mesh=mesh,
        out_type=[jax.ShapeDtypeStruct((2, n_ent, HALF), jnp.float32),
                  jax.ShapeDtypeStruct((n_ent,), jnp.float32)],
        scratch_types=_buf_types + _buf_types + [
            pltpu.VMEM_SHARED((n_ent, HALF), jnp.float32),
            pltpu.VMEM_SHARED((n_ent,), jnp.float32),
        ],
    )
    def stage_b(h_hbm, r_hbm, t_hbm, q1_hbm, q2_hbm, q3_hbm,
                t2_hbm, t3_hbm, s_hbm, d_hbm, *scr):
        buf0 = scr[:NB]
        buf1 = scr[NB:2 * NB]
        acc, accd = scr[2 * NB:]
        zdv = buf0[8]
        zv = buf0[9]
        cid = lax.axis_index("c")
        sid = lax.axis_index("s")

        zvec = jnp.zeros((L,), jnp.float32)

        def _zrow(i, _):
            for j in range(HALF // L):
                zv[i, pl.ds(j * L, L)] = zvec
            return 0

        lax.fori_loop(0, RB, _zrow, 0)
        for j in range(C // L):
            zdv[pl.ds(j * L, L)] = zvec
        for b in range(nrb):
            m = sid + NS * b

            @pl.when(m < nblk)
            def _():
                pltpu.sync_copy(zv, acc.at[pl.ds(m * RB, RB)])
                pltpu.sync_copy(zdv, accd.at[pl.ds(m * RB, RB)])
        plsc.subcore_barrier()

        base0 = sid * epw
        coff = cid * n_ent

        def _copies(buf):
            hv, rv, tv, rfv, tfv, q1g, q2g, q3g = buf[:8]
            semq, sem2, sem3 = buf[11:14]
            return (pltpu.make_async_copy(t2_hbm.at[rfv], buf[9], sem2),
                    pltpu.make_async_copy(t3_hbm.at[tfv], buf[10], sem3),
                    pltpu.make_async_copy(q1_hbm.at[hv], q1g, semq),
                    pltpu.make_async_copy(q2_hbm.at[rv], q2g, semq),
                    pltpu.make_async_copy(q3_hbm.at[tv], q3g, semq))

        def _prefetch(buf, k):
            hv, rv, tv, rfv, tfv = buf[:5]
            b = base0 + k * C
            pltpu.sync_copy(h_hbm.at[pl.ds(b, C)], hv)
            pltpu.sync_copy(r_hbm.at[pl.ds(b, C)], rv)
            pltpu.sync_copy(t_hbm.at[pl.ds(b, C)], tv)
            for j in range(C // L):
                sl = pl.ds(j * L, L)
                rfv[sl] = rv[sl] + coff
                tfv[sl] = tv[sl] + coff
            for cp in _copies(buf):
                cp.start()

        def _process(buf):
            hv, rv, tv, rfv, tfv, q1g, q2g, q3g, wv, r2v, r3v = buf[:11]
            cp2, cp3, cq1, cq2, cq3 = _copies(buf)
            cq1.wait()
            cq2.wait()
            cq3.wait()
            for j in range(C // L):
                sl = pl.ds(j * L, L)
                s = q1g[sl] + q2g[sl] + q3g[sl]
                wv[sl] = jnp.exp(jnp.maximum(s, 0.01 * s))
            cp2.wait()
            cp3.wait()

            def _grp(g, _):
                w16 = wv[pl.ds(g * L, L)]
                for lane in range(L):
                    e = g * L + lane
                    ws = w16[lane]
                    for j in range(HALF // L):
                        sl = pl.ds(j * L, L)
                        r2v[e, sl] = (r2v[e, sl] + r3v[e, sl]) * ws
                return 0

            lax.fori_loop(0, C // L, _grp, 0)
            pltpu.sync_copy(r2v, acc.at[hv], add=True)

            @pl.when(cid == 0)
            def _():
                pltpu.sync_copy(wv, accd.at[hv], add=True)

        _prefetch(buf0, 0)

        def _pair(i, _):
            k = 2 * i
            _prefetch(buf1, k + 1)
            _process(buf0)
            _prefetch(buf0, k + 2)
            _process(buf1)
            return 0

        lax.fori_loop(0, (nch - 1) // 2, _pair, 0)
        _process(buf0)
        plsc.subcore_barrier()
        for b in range(nrb):
            m = sid + NS * b

            @pl.when(m < nblk)
            def _():
                s0 = m * RB
                pltpu.sync_copy(acc.at[pl.ds(s0, RB)],
                                s_hbm.at[cid, pl.ds(s0, RB)])

                @pl.when(cid == 0)
                def _():
                    pltpu.sync_copy(accd.at[pl.ds(s0, RB)], zdv)
                    pltpu.sync_copy(zdv, d_hbm.at[pl.ds(s0, RB)])

    return stage_b


def _c_body(s_ref, d_ref, p1_ref, o_ref):
    d = d_ref[...]
    s = jnp.concatenate([s_ref[0], s_ref[1]], axis=1)
    numer = d * p1_ref[...] + s
    x = numer / jnp.maximum(d, 1e-12)
    o_ref[...] = jnp.where(x > 0, x, jnp.exp(jnp.minimum(x, 0.0)) - 1.0)


def _stage_c(sacc, denom2d, p1):
    n, d = p1.shape
    bn = 2000
    return pl.pallas_call(
        _c_body,
        grid=(n // bn,),
        in_specs=[
            pl.BlockSpec((2, bn, HALF), lambda i: (0, i, 0)),
            pl.BlockSpec((bn, 1), lambda i: (i, 0)),
            pl.BlockSpec((bn, d), lambda i: (i, 0)),
        ],
        out_specs=pl.BlockSpec((bn, d), lambda i: (i, 0)),
        out_shape=jax.ShapeDtypeStruct((n, d), jnp.float32),
    )(sacc, denom2d, p1)


def kernel(triplets, ent_embed, rel_embed, adj, a_w, a_b, a2_w, a2_b):
    del adj
    n, d = ent_embed.shape
    e = triplets.shape[1]
    a_b2d = a_b.reshape(1, d)
    p1, t2, t3, q = _stage_a(ent_embed, rel_embed, a_w, a_b2d, a2_w)
    q1 = q[:, 0] + a2_b[0]
    t2f = t2.reshape(2 * n, HALF)
    t3f = t3.reshape(2 * n, HALF)
    sacc, denom = _make_stage_b(n, e)(triplets[0], triplets[1], triplets[2],
                                      q1, q[:, 1], q[:, 2], t2f, t3f)
    return _stage_c(sacc, denom.reshape(n, 1), p1)


# speedup vs baseline: 4.8176x; 1.0955x over previous
import functools

import jax
import jax.numpy as jnp
from jax import lax
from jax.experimental import pallas as pl
from jax.experimental.pallas import tpu as pltpu
from jax.experimental.pallas import tpu_sc as plsc

HALF = 128


def _a_body(ent_ref, rel_ref, aw_ref, ab_ref, a2_ref, p1_ref, t2_ref, t3_ref,
            q_ref):
    e = ent_ref[...]
    r = rel_ref[...]
    aw = aw_ref[...]
    hi = lax.Precision.HIGHEST
    dn = (((1,), (1,)), ((), ()))
    c1 = lax.dot_general(e, aw[:, 0:256], dn, precision=hi,
                         preferred_element_type=jnp.float32) + ab_ref[...]
    c2 = lax.dot_general(r, aw[:, 256:512], dn, precision=hi,
                         preferred_element_type=jnp.float32)
    c3 = lax.dot_general(e, aw[:, 512:768], dn, precision=hi,
                         preferred_element_type=jnp.float32)
    a2 = a2_ref[...]
    q1 = lax.dot_general(c1, a2, dn, precision=hi,
                         preferred_element_type=jnp.float32)
    q2 = lax.dot_general(c2, a2, dn, precision=hi,
                         preferred_element_type=jnp.float32)
    q3 = lax.dot_general(c3, a2, dn, precision=hi,
                         preferred_element_type=jnp.float32)
    q_ref[...] = jnp.concatenate([q1, q2, q3, jnp.zeros_like(q1)], axis=1)
    p1_ref[...] = c1
    t2_ref[0] = c2[:, :HALF]
    t2_ref[1] = c2[:, HALF:]
    t3_ref[0] = c3[:, :HALF]
    t3_ref[1] = c3[:, HALF:]


def _stage_a(ent, rel, a_w, a_b2d, a2_w):
    n, d = ent.shape
    bn = 2000
    grid = (n // bn,)
    return pl.pallas_call(
        _a_body,
        grid=grid,
        in_specs=[
            pl.BlockSpec((bn, d), lambda i: (i, 0)),
            pl.BlockSpec((bn, d), lambda i: (i, 0)),
            pl.BlockSpec((d, 3 * d), lambda i: (0, 0)),
            pl.BlockSpec((1, d), lambda i: (0, 0)),
            pl.BlockSpec((1, d), lambda i: (0, 0)),
        ],
        out_specs=[
            pl.BlockSpec((bn, d), lambda i: (i, 0)),
            pl.BlockSpec((2, bn, HALF), lambda i: (0, i, 0)),
            pl.BlockSpec((2, bn, HALF), lambda i: (0, i, 0)),
            pl.BlockSpec((bn, 4), lambda i: (i, 0)),
        ],
        out_shape=[
            jax.ShapeDtypeStruct((n, d), jnp.float32),
            jax.ShapeDtypeStruct((2, n, HALF), jnp.float32),
            jax.ShapeDtypeStruct((2, n, HALF), jnp.float32),
            jax.ShapeDtypeStruct((n, 4), jnp.float32),
        ],
    )(ent, rel, a_w, a_b2d, a2_w)


def _make_stage_b(n_ent, n_edges):
    NC, NS, L = 2, 16, 16
    C = 80
    PE = 400
    PC = PE // C
    assert PC % 2 == 1
    epw = n_edges // NS
    NP = epw // PE
    RB = 80
    nblk = n_ent // RB
    nrb = -(-nblk // NS)
    mesh = plsc.VectorSubcoreMesh(core_axis_name="c", subcore_axis_name="s")

    _buf_types = [
        pltpu.VMEM((C,), jnp.int32),
        pltpu.VMEM((C,), jnp.int32),
        pltpu.VMEM((C,), jnp.int32),
        pltpu.VMEM((C,), jnp.float32),
        pltpu.VMEM((C,), jnp.float32),
        pltpu.VMEM((C,), jnp.float32),
        pltpu.VMEM((C,), jnp.float32),
        pltpu.VMEM((C, HALF), jnp.float32),
        pltpu.VMEM((C, HALF), jnp.float32),
        pltpu.SemaphoreType.DMA,
        pltpu.SemaphoreType.DMA,
        pltpu.SemaphoreType.DMA,
        pltpu.SemaphoreType.DMA,
    ]
    NB = len(_buf_types)

    @functools.partial(
        pl.kernel,
        mesh=mesh,
        out_type=[jax.ShapeDtypeStruct((2, n_ent, HALF), jnp.float32),
                  jax.ShapeDtypeStruct((n_ent,), jnp.float32)],
        scratch_types=_buf_types + _buf_types + [
            pltpu.VMEM((PE,), jnp.int32),
            pltpu.VMEM((PE,), jnp.int32),
            pltpu.VMEM((PE,), jnp.int32),
            pltpu.VMEM_SHARED((n_ent, HALF), jnp.float32),
            pltpu.VMEM_SHARED((n_ent,), jnp.float32),
        ],
    )
    def stage_b(h_hbm, r_hbm, t_hbm, q1_hbm, q2d_hbm, q3d_hbm,
                t2_hbm, t3_hbm, s_hbm, d_hbm, *scr):
        buf0 = scr[:NB]
        buf1 = scr[NB:2 * NB]
        hb, rb, tb, acc, accd = scr[2 * NB:]
        cid = lax.axis_index("c")
        sid = lax.axis_index("s")
        zvec = jnp.zeros((L,), jnp.float32)
        zivec = jnp.zeros((L,), jnp.int32)

        def _zrow(i, _):
            for j in range(HALF // L):
                buf0[7][i, pl.ds(j * L, L)] = zvec
            return 0

        lax.fori_loop(0, RB, _zrow, 0)
        for buf in (buf0, buf1):
            for j in range(C // L):
                sl = pl.ds(j * L, L)
                buf[6][sl] = zvec
                buf[0][sl] = zivec
        for b in range(nrb):
            m = sid + NS * b

            @pl.when(m < nblk)
            def _():
                pltpu.sync_copy(buf0[7], acc.at[pl.ds(m * RB, RB)])
                pltpu.sync_copy(buf0[6], accd.at[pl.ds(m * RB, RB)])
        plsc.subcore_barrier()
        for buf in (buf0, buf1):
            pltpu.async_copy(buf[6], accd.at[buf[0]], buf[12], add=True)

        base0 = sid * epw
        coff = cid * n_ent

        def _copies(buf):
            return (pltpu.make_async_copy(t2_hbm.at[buf[1]], buf[7], buf[10]),
                    pltpu.make_async_copy(t3_hbm.at[buf[2]], buf[8], buf[11]),
                    pltpu.make_async_copy(q1_hbm.at[buf[0]], buf[3], buf[9]),
                    pltpu.make_async_copy(q2d_hbm.at[buf[1]], buf[4], buf[9]),
                    pltpu.make_async_copy(q3d_hbm.at[buf[2]], buf[5], buf[9]))

        def _prefetch(buf, off):
            hv, rfv, tfv = buf[0], buf[1], buf[2]
            pltpu.make_async_copy(buf[6], accd.at[hv], buf[12]).wait()
            for j in range(C // L):
                dl = pl.ds(j * L, L)
                sl = pl.ds(off + j * L, L)
                hv[dl] = hb[sl]
                rfv[dl] = rb[sl] + coff
                tfv[dl] = tb[sl] + coff
            for cp in _copies(buf):
                cp.start()

        def _process(buf):
            q1g, q2g, q3g, wv, r2v, r3v = buf[3:9]
            cp2, cp3, cq1, cq2, cq3 = _copies(buf)
            cq1.wait()
            cq2.wait()
            cq3.wait()
            for j in range(C // L):
                sl = pl.ds(j * L, L)
                s = q1g[sl] + q2g[sl] + q3g[sl]
                wv[sl] = jnp.exp(jnp.maximum(s, 0.01 * s))
            cp2.wait()
            cp3.wait()

            def _grp(g, _):
                w16 = wv[pl.ds(g * L, L)]
                for lane in range(L):
                    e = g * L + lane
                    ws = w16[lane]
                    for j in range(HALF // L):
                        sl = pl.ds(j * L, L)
                        r2v[e, sl] = (r2v[e, sl] + r3v[e, sl]) * ws
                return 0

            lax.fori_loop(0, C // L, _grp, 0)
            pltpu.sync_copy(r2v, acc.at[buf[0]], add=True)
            pltpu.async_copy(wv, accd.at[buf[0]], buf[12], add=True)

        def _phase(p, _):
            pb = base0 + p * PE
            pltpu.sync_copy(h_hbm.at[pl.ds(pb, PE)], hb)
            pltpu.sync_copy(r_hbm.at[pl.ds(pb, PE)], rb)
            pltpu.sync_copy(t_hbm.at[pl.ds(pb, PE)], tb)
            _prefetch(buf0, 0)

            def _pair(i, _):
                _prefetch(buf1, (2 * i + 1) * C)
                _process(buf0)
                _prefetch(buf0, (2 * i + 2) * C)
                _process(buf1)
                return 0

            lax.fori_loop(0, (PC - 1) // 2, _pair, 0)
            _process(buf0)
            return 0

        lax.fori_loop(0, NP, _phase, 0)
        for buf in (buf0, buf1):
            pltpu.make_async_copy(buf[6], accd.at[buf[0]], buf[12]).wait()
        plsc.subcore_barrier()
        for b in range(nrb):
            m = sid + NS * b

            @pl.when(m < nblk)
            def _():
                s0 = m * RB
                pltpu.sync_copy(acc.at[pl.ds(s0, RB)],
                                s_hbm.at[cid, pl.ds(s0, RB)])

                @pl.when(cid == 0)
                def _():
                    pltpu.sync_copy(accd.at[pl.ds(s0, RB)], buf0[6])
                    pltpu.sync_copy(buf0[6], d_hbm.at[pl.ds(s0, RB)])

    return stage_b


def _c_body(s_ref, d_ref, p1_ref, o_ref):
    d = d_ref[...]
    s = jnp.concatenate([s_ref[0], s_ref[1]], axis=1)
    numer = d * p1_ref[...] + s
    x = numer / jnp.maximum(d, 1e-12)
    o_ref[...] = jnp.where(x > 0, x, jnp.exp(jnp.minimum(x, 0.0)) - 1.0)


def _stage_c(sacc, denom2d, p1):
    n, d = p1.shape
    bn = 2000
    return pl.pallas_call(
        _c_body,
        grid=(n // bn,),
        in_specs=[
            pl.BlockSpec((2, bn, HALF), lambda i: (0, i, 0)),
            pl.BlockSpec((bn, 1), lambda i: (i, 0)),
            pl.BlockSpec((bn, d), lambda i: (i, 0)),
        ],
        out_specs=pl.BlockSpec((bn, d), lambda i: (i, 0)),
        out_shape=jax.ShapeDtypeStruct((n, d), jnp.float32),
    )(sacc, denom2d, p1)


def kernel(triplets, ent_embed, rel_embed, adj, a_w, a_b, a2_w, a2_b):
    del adj
    n, d = ent_embed.shape
    e = triplets.shape[1]
    a_b2d = a_b.reshape(1, d)
    p1, t2, t3, q = _stage_a(ent_embed, rel_embed, a_w, a_b2d, a2_w)
    q1 = q[:, 0] + a2_b[0]
    q2d = jnp.concatenate([q[:, 1], q[:, 1]])
    q3d = jnp.concatenate([q[:, 2], q[:, 2]])
    t2f = t2.reshape(2 * n, HALF)
    t3f = t3.reshape(2 * n, HALF)
    sacc, denom = _make_stage_b(n, e)(triplets[0], triplets[1], triplets[2],
                                      q1, q2d, q3d, t2f, t3f)
    return _stage_c(sacc, denom.reshape(n, 1), p1)


# speedup vs baseline: 5.5106x; 1.1439x over previous
import functools

import jax
import jax.numpy as jnp
from jax import lax
from jax.experimental import pallas as pl
from jax.experimental.pallas import tpu as pltpu
from jax.experimental.pallas import tpu_sc as plsc

HALF = 128


def _a_body(ent_ref, rel_ref, aw_ref, ab_ref, a2_ref, p1_ref, t2_ref, t3_ref,
            q_ref):
    e = ent_ref[...]
    r = rel_ref[...]
    aw = aw_ref[...]
    hi = lax.Precision.HIGHEST
    dn = (((1,), (1,)), ((), ()))
    c1 = lax.dot_general(e, aw[:, 0:256], dn, precision=hi,
                         preferred_element_type=jnp.float32) + ab_ref[...]
    c2 = lax.dot_general(r, aw[:, 256:512], dn, precision=hi,
                         preferred_element_type=jnp.float32)
    c3 = lax.dot_general(e, aw[:, 512:768], dn, precision=hi,
                         preferred_element_type=jnp.float32)
    a2 = a2_ref[...]
    q1 = lax.dot_general(c1, a2, dn, precision=hi,
                         preferred_element_type=jnp.float32)
    q2 = lax.dot_general(c2, a2, dn, precision=hi,
                         preferred_element_type=jnp.float32)
    q3 = lax.dot_general(c3, a2, dn, precision=hi,
                         preferred_element_type=jnp.float32)
    q_ref[...] = jnp.concatenate([q1, q2, q3, jnp.zeros_like(q1)], axis=1)
    p1_ref[...] = c1
    t2_ref[0] = c2[:, :HALF]
    t2_ref[1] = c2[:, HALF:]
    t3_ref[0] = c3[:, :HALF]
    t3_ref[1] = c3[:, HALF:]


def _stage_a(ent, rel, a_w, a_b2d, a2_w):
    n, d = ent.shape
    bn = 2000
    grid = (n // bn,)
    return pl.pallas_call(
        _a_body,
        grid=grid,
        in_specs=[
            pl.BlockSpec((bn, d), lambda i: (i, 0)),
            pl.BlockSpec((bn, d), lambda i: (i, 0)),
            pl.BlockSpec((d, 3 * d), lambda i: (0, 0)),
            pl.BlockSpec((1, d), lambda i: (0, 0)),
            pl.BlockSpec((1, d), lambda i: (0, 0)),
        ],
        out_specs=[
            pl.BlockSpec((bn, d), lambda i: (i, 0)),
            pl.BlockSpec((2, bn, HALF), lambda i: (0, i, 0)),
            pl.BlockSpec((2, bn, HALF), lambda i: (0, i, 0)),
            pl.BlockSpec((bn, 4), lambda i: (i, 0)),
        ],
        out_shape=[
            jax.ShapeDtypeStruct((n, d), jnp.float32),
            jax.ShapeDtypeStruct((2, n, HALF), jnp.float32),
            jax.ShapeDtypeStruct((2, n, HALF), jnp.float32),
            jax.ShapeDtypeStruct((n, 4), jnp.float32),
        ],
    )(ent, rel, a_w, a_b2d, a2_w)


def _make_stage_b(n_ent, n_edges):
    NC, NS, L = 2, 16, 16
    C = 80
    PE = 400
    PC = PE // C
    assert PC % 2 == 1
    epw = n_edges // NS
    NP = epw // PE
    RB = 80
    nblk = n_ent // RB
    nrb = -(-nblk // NS)
    mesh = plsc.VectorSubcoreMesh(core_axis_name="c", subcore_axis_name="s")

    _buf_types = [
        pltpu.VMEM((C,), jnp.int32),
        pltpu.VMEM((C,), jnp.int32),
        pltpu.VMEM((C,), jnp.int32),
        pltpu.VMEM((C,), jnp.float32),
        pltpu.VMEM((C,), jnp.float32),
        pltpu.VMEM((C,), jnp.float32),
        pltpu.VMEM((C,), jnp.float32),
        pltpu.VMEM((C, HALF), jnp.float32),
        pltpu.VMEM((C, HALF), jnp.float32),
        pltpu.SemaphoreType.DMA,
        pltpu.SemaphoreType.DMA,
        pltpu.SemaphoreType.DMA,
        pltpu.SemaphoreType.DMA,
    ]
    NB = len(_buf_types)

    @functools.partial(
        pl.kernel,
        mesh=mesh,
        out_type=[jax.ShapeDtypeStruct((2, n_ent, HALF), jnp.float32),
                  jax.ShapeDtypeStruct((n_ent,), jnp.float32)],
        scratch_types=_buf_types + _buf_types + [
            pltpu.VMEM((PE,), jnp.int32),
            pltpu.VMEM((PE,), jnp.int32),
            pltpu.VMEM((PE,), jnp.int32),
            pltpu.VMEM_SHARED((n_ent, HALF), jnp.float32),
            pltpu.VMEM_SHARED((n_ent,), jnp.float32),
        ],
    )
    def stage_b(h_hbm, r_hbm, t_hbm, q1_hbm, q2d_hbm, q3d_hbm,
                t2_hbm, t3_hbm, s_hbm, d_hbm, *scr):
        buf0 = scr[:NB]
        buf1 = scr[NB:2 * NB]
        hb, rb, tb, acc, accd = scr[2 * NB:]
        cid = lax.axis_index("c")
        sid = lax.axis_index("s")
        zvec = jnp.zeros((L,), jnp.float32)
        zivec = jnp.zeros((L,), jnp.int32)

        def _zrow(i, _):
            for j in range(HALF // L):
                buf0[7][i, pl.ds(j * L, L)] = zvec
            return 0

        lax.fori_loop(0, RB, _zrow, 0)
        for buf in (buf0, buf1):
            for j in range(C // L):
                sl = pl.ds(j * L, L)
                buf[6][sl] = zvec
                buf[0][sl] = zivec
        for b in range(nrb):
            m = sid + NS * b

            @pl.when(m < nblk)
            def _():
                pltpu.sync_copy(buf0[7], acc.at[pl.ds(m * RB, RB)])
                pltpu.sync_copy(buf0[6], accd.at[pl.ds(m * RB, RB)])
        plsc.subcore_barrier()
        for buf in (buf0, buf1):
            pltpu.async_copy(buf[6], accd.at[buf[0]], buf[12], add=True)

        base0 = sid * epw
        coff = cid * n_ent

        def _copies(buf):
            return (pltpu.make_async_copy(t2_hbm.at[buf[1]], buf[7], buf[10]),
                    pltpu.make_async_copy(t3_hbm.at[buf[2]], buf[8], buf[11]),
                    pltpu.make_async_copy(q1_hbm.at[buf[0]], buf[3], buf[9]),
                    pltpu.make_async_copy(q2d_hbm.at[buf[1]], buf[4], buf[9]),
                    pltpu.make_async_copy(q3d_hbm.at[buf[2]], buf[5], buf[9]))

        def _prefetch(buf, off):
            hv, rfv, tfv = buf[0], buf[1], buf[2]
            pltpu.make_async_copy(buf[6], accd.at[hv], buf[12]).wait()
            for j in range(C // L):
                dl = pl.ds(j * L, L)
                sl = pl.ds(off + j * L, L)
                hv[dl] = hb[sl]
                rfv[dl] = rb[sl] + coff
                tfv[dl] = tb[sl] + coff
            for cp in _copies(buf):
                cp.start()

        def _process(buf):
            q1g, q2g, q3g, wv, r2v, r3v = buf[3:9]
            cp2, cp3, cq1, cq2, cq3 = _copies(buf)
            cq1.wait()
            cq2.wait()
            cq3.wait()
            for j in range(C // L):
                sl = pl.ds(j * L, L)
                s = q1g[sl] + q2g[sl] + q3g[sl]
                wv[sl] = jnp.exp(jnp.maximum(s, 0.01 * s))
            cp2.wait()
            cp3.wait()

            def _grp(g, _):
                w16 = wv[pl.ds(g * L, L)]
                for lane in range(L):
                    e = g * L + lane
                    wsv = w16[jnp.full((L,), lane, jnp.int32)]
                    for j in range(HALF // L):
                        sl = pl.ds(j * L, L)
                        r2v[e, sl] = (r2v[e, sl] + r3v[e, sl]) * wsv
                return 0

            lax.fori_loop(0, C // L, _grp, 0)
            pltpu.sync_copy(r2v, acc.at[buf[0]], add=True)
            pltpu.async_copy(wv, accd.at[buf[0]], buf[12], add=True)

        def _phase(p, _):
            pb = base0 + p * PE
            pltpu.sync_copy(h_hbm.at[pl.ds(pb, PE)], hb)
            pltpu.sync_copy(r_hbm.at[pl.ds(pb, PE)], rb)
            pltpu.sync_copy(t_hbm.at[pl.ds(pb, PE)], tb)
            _prefetch(buf0, 0)

            def _pair(i, _):
                _prefetch(buf1, (2 * i + 1) * C)
                _process(buf0)
                _prefetch(buf0, (2 * i + 2) * C)
                _process(buf1)
                return 0

            lax.fori_loop(0, (PC - 1) // 2, _pair, 0)
            _process(buf0)
            return 0

        lax.fori_loop(0, NP, _phase, 0)
        for buf in (buf0, buf1):
            pltpu.make_async_copy(buf[6], accd.at[buf[0]], buf[12]).wait()
        plsc.subcore_barrier()
        for b in range(nrb):
            m = sid + NS * b

            @pl.when(m < nblk)
            def _():
                s0 = m * RB
                pltpu.sync_copy(acc.at[pl.ds(s0, RB)],
                                s_hbm.at[cid, pl.ds(s0, RB)])

                @pl.when(cid == 0)
                def _():
                    pltpu.sync_copy(accd.at[pl.ds(s0, RB)], buf0[6])
                    pltpu.sync_copy(buf0[6], d_hbm.at[pl.ds(s0, RB)])

    return stage_b


def _c_body(s_ref, d_ref, p1_ref, o_ref):
    d = d_ref[...]
    s = jnp.concatenate([s_ref[0], s_ref[1]], axis=1)
    numer = d * p1_ref[...] + s
    x = numer / jnp.maximum(d, 1e-12)
    o_ref[...] = jnp.where(x > 0, x, jnp.exp(jnp.minimum(x, 0.0)) - 1.0)


def _stage_c(sacc, denom2d, p1):
    n, d = p1.shape
    bn = 2000
    return pl.pallas_call(
        _c_body,
        grid=(n // bn,),
        in_specs=[
            pl.BlockSpec((2, bn, HALF), lambda i: (0, i, 0)),
            pl.BlockSpec((bn, 1), lambda i: (i, 0)),
            pl.BlockSpec((bn, d), lambda i: (i, 0)),
        ],
        out_specs=pl.BlockSpec((bn, d), lambda i: (i, 0)),
        out_shape=jax.ShapeDtypeStruct((n, d), jnp.float32),
    )(sacc, denom2d, p1)


def kernel(triplets, ent_embed, rel_embed, adj, a_w, a_b, a2_w, a2_b):
    del adj
    n, d = ent_embed.shape
    e = triplets.shape[1]
    a_b2d = a_b.reshape(1, d)
    p1, t2, t3, q = _stage_a(ent_embed, rel_embed, a_w, a_b2d, a2_w)
    q1 = q[:, 0] + a2_b[0]
    q2d = jnp.concatenate([q[:, 1], q[:, 1]])
    q3d = jnp.concatenate([q[:, 2], q[:, 2]])
    t2f = t2.reshape(2 * n, HALF)
    t3f = t3.reshape(2 * n, HALF)
    sacc, denom = _make_stage_b(n, e)(triplets[0], triplets[1], triplets[2],
                                      q1, q2d, q3d, t2f, t3f)
    return _stage_c(sacc, denom.reshape(n, 1), p1)
